# baseline (speedup 1.0000x reference)
import jax
import jax.numpy as jnp
from jax import lax
from jax.experimental import pallas as pl
from jax.experimental.pallas import tpu as pltpu
from jax.experimental.pallas import tpu_sc as plsc

_NUM_EXPERTS = 64
_TOP_K = 8
_NC = 2
_NS = 16
_LANES = 16
_TILES = _NC * _NS
_RB = 512


def _tc_kernel(x_ref, wt_ref, logits_ref, lt_ref):
    logits = jnp.dot(x_ref[...], wt_ref[...],
                     preferred_element_type=jnp.float32)
    logits_ref[...] = logits
    lt_ref[...] = logits.T


def _sc_body(lt_hbm, wt_hbm, it_hbm, lbuf, wbuf, ibuf):
    wid = lax.axis_index("s") * _NC + lax.axis_index("c")
    total = lt_hbm.shape[1]
    rows_per_tile = total // _TILES
    base = wid * rows_per_tile
    n_sub = rows_per_tile // _RB

    @pl.loop(0, n_sub)
    def _sub(sub):
        col0 = base + sub * _RB
        pltpu.sync_copy(lt_hbm.at[:, pl.ds(col0, _RB)], lbuf)

        @plsc.parallel_loop(0, _RB // _LANES)
        def _chunk(c):
            sl = pl.ds(c * _LANES, _LANES)
            t = [jnp.full((_LANES,), -jnp.inf, jnp.float32)] * _TOP_K
            ti = [jnp.zeros((_LANES,), jnp.int32)] * _TOP_K
            for e in range(_NUM_EXPERTS):
                v = lbuf[e, sl]
                gt = v > t[7]
                t[7] = jnp.where(gt, v, t[7])
                ti[7] = jnp.where(gt, e, ti[7])
                for j in range(_TOP_K - 1, 0, -1):
                    sw = t[j] > t[j - 1]
                    a, b = t[j - 1], t[j]
                    t[j - 1] = jnp.where(sw, b, a)
                    t[j] = jnp.where(sw, a, b)
                    ai, bi = ti[j - 1], ti[j]
                    ti[j - 1] = jnp.where(sw, bi, ai)
                    ti[j] = jnp.where(sw, ai, bi)
            ex = [jnp.exp(t[j] - t[0]) for j in range(_TOP_K)]
            ssum = ((ex[0] + ex[1]) + (ex[2] + ex[3])) + \
                   ((ex[4] + ex[5]) + (ex[6] + ex[7]))
            for j in range(_TOP_K):
                wbuf[j, sl] = ex[j] / ssum
                ibuf[j, sl] = ti[j]

        pltpu.sync_copy(wbuf, wt_hbm.at[:, pl.ds(col0, _RB)])
        pltpu.sync_copy(ibuf, it_hbm.at[:, pl.ds(col0, _RB)])


def _sc_topk(logits_t):
    total = logits_t.shape[1]
    mesh = plsc.VectorSubcoreMesh(
        core_axis_name="c", subcore_axis_name="s",
        num_cores=_NC, num_subcores=_NS)
    w_t, i_t = pl.kernel(
        _sc_body,
        out_type=[
            jax.ShapeDtypeStruct((_TOP_K, total), jnp.float32),
            jax.ShapeDtypeStruct((_TOP_K, total), jnp.int32),
        ],
        mesh=mesh,
        scratch_types=[
            pltpu.VMEM((_NUM_EXPERTS, _RB), jnp.float32),
            pltpu.VMEM((_TOP_K, _RB), jnp.float32),
            pltpu.VMEM((_TOP_K, _RB), jnp.int32),
        ],
    )(logits_t)
    return w_t.T, i_t.T


def kernel(hidden_states, router_weight):
    b, s, h = hidden_states.shape
    ne = router_weight.shape[0]
    x = hidden_states.reshape(b * s, h)
    wt = router_weight.T
    total = b * s
    bt = 512
    logits, logits_t = pl.pallas_call(
        _tc_kernel,
        grid=(total // bt,),
        in_specs=[
            pl.BlockSpec((bt, h), lambda i: (i, 0)),
            pl.BlockSpec((h, ne), lambda i: (0, 0)),
        ],
        out_specs=[
            pl.BlockSpec((bt, ne), lambda i: (i, 0)),
            pl.BlockSpec((ne, bt), lambda i: (0, i)),
        ],
        out_shape=[
            jax.ShapeDtypeStruct((total, ne), jnp.float32),
            jax.ShapeDtypeStruct((ne, total), jnp.float32),
        ],
    )(x, wt)
    w, idx = _sc_topk(logits_t)
    return (w, idx, logits)


# speedup vs baseline: 1.0942x; 1.0942x over previous
import jax
import jax.numpy as jnp
from jax import lax
from jax.experimental import pallas as pl
from jax.experimental.pallas import tpu as pltpu
from jax.experimental.pallas import tpu_sc as plsc

_NUM_EXPERTS = 64
_TOP_K = 8
_NC = 2
_NS = 16
_LANES = 16
_TILES = _NC * _NS
_RB = 512


def _tc_kernel(x_ref, wt_ref, logits_ref, lt_ref):
    logits = jnp.dot(x_ref[...], wt_ref[...],
                     preferred_element_type=jnp.float32)
    logits_ref[...] = logits
    lt_ref[...] = logits.T


def _sc_body(lt_hbm, wt_hbm, it_hbm, lbuf, wbuf, ibuf):
    wid = lax.axis_index("s") * _NC + lax.axis_index("c")
    total = lt_hbm.shape[1]
    rows_per_tile = total // _TILES
    base = wid * rows_per_tile
    n_sub = rows_per_tile // _RB

    @pl.loop(0, n_sub)
    def _sub(sub):
        col0 = base + sub * _RB
        pltpu.sync_copy(lt_hbm.at[:, pl.ds(col0, _RB)], lbuf)

        @plsc.parallel_loop(0, _RB // _LANES)
        def _chunk(c):
            sl = pl.ds(c * _LANES, _LANES)
            t = [jnp.full((_LANES,), -jnp.inf, jnp.float32)] * _TOP_K
            ti = [jnp.zeros((_LANES,), jnp.int32)] * _TOP_K
            for e in range(_NUM_EXPERTS):
                v = lbuf[e, sl]
                gt = v > t[7]
                t[7] = jnp.where(gt, v, t[7])
                ti[7] = jnp.where(gt, e, ti[7])
                for j in range(_TOP_K - 1, 0, -1):
                    sw = t[j] > t[j - 1]
                    a, b = t[j - 1], t[j]
                    t[j - 1] = jnp.where(sw, b, a)
                    t[j] = jnp.where(sw, a, b)
                    ai, bi = ti[j - 1], ti[j]
                    ti[j - 1] = jnp.where(sw, bi, ai)
                    ti[j] = jnp.where(sw, ai, bi)
            ex = [jnp.exp(t[j] - t[0]) for j in range(_TOP_K)]
            ssum = ((ex[0] + ex[1]) + (ex[2] + ex[3])) + \
                   ((ex[4] + ex[5]) + (ex[6] + ex[7]))
            for j in range(_TOP_K):
                wbuf[j, sl] = ex[j] / ssum
                ibuf[j, sl] = ti[j]

        pltpu.sync_copy(wbuf, wt_hbm.at[:, pl.ds(col0, _RB)])
        pltpu.sync_copy(ibuf, it_hbm.at[:, pl.ds(col0, _RB)])


def _sc_topk(logits_t):
    total = logits_t.shape[1]
    mesh = plsc.VectorSubcoreMesh(
        core_axis_name="c", subcore_axis_name="s",
        num_cores=_NC, num_subcores=_NS)
    w_t, i_t = pl.kernel(
        _sc_body,
        out_type=[
            jax.ShapeDtypeStruct((_TOP_K, total), jnp.float32),
            jax.ShapeDtypeStruct((_TOP_K, total), jnp.int32),
        ],
        mesh=mesh,
        scratch_types=[
            pltpu.VMEM((_NUM_EXPERTS, _RB), jnp.float32),
            pltpu.VMEM((_TOP_K, _RB), jnp.float32),
            pltpu.VMEM((_TOP_K, _RB), jnp.int32),
        ],
    )(logits_t)
    return w_t.T, i_t.T


def kernel(hidden_states, router_weight):
    b, s, h = hidden_states.shape
    ne = router_weight.shape[0]
    x = hidden_states.reshape(b * s, h)
    wt = router_weight.T
    total = b * s
    bt = 1024
    logits, logits_t = pl.pallas_call(
        _tc_kernel,
        grid=(total // bt,),
        in_specs=[
            pl.BlockSpec((bt, h), lambda i: (i, 0)),
            pl.BlockSpec((h, ne), lambda i: (0, 0)),
        ],
        out_specs=[
            pl.BlockSpec((bt, ne), lambda i: (i, 0)),
            pl.BlockSpec((ne, bt), lambda i: (0, i)),
        ],
        out_shape=[
            jax.ShapeDtypeStruct((total, ne), jnp.float32),
            jax.ShapeDtypeStruct((ne, total), jnp.float32),
        ],
    )(x, wt)
    w, idx = _sc_topk(logits_t)
    return (w, idx, logits)
